# baseline (speedup 1.0000x reference)
import functools

import jax
import jax.numpy as jnp
from jax.experimental import pallas as pl
from jax.experimental.pallas import tpu as pltpu

NUM_GENES = 18211
INTER_DIM = 1024
LATENT_DIM = 128
BATCH = 64

KBLK = 2048
NK = (NUM_GENES + KBLK - 1) // KBLK


def _data_blk(s):
    return jax.lax.rem(s + NK - 1, NK)


def _mlp_kernel(x_ref, w1a_ref, w1b_ref, b1_ref, w2_ref, b2_ref, z_ref,
                acc_ref):
    s = pl.program_id(0)
    x_blk = x_ref[...]

    def partial_dot(w_blk):
        @pl.when(s == 0)
        def _first():
            base = (NK - 1) * KBLK
            col_ids = jax.lax.broadcasted_iota(jnp.int32, (BATCH, KBLK), 1)
            xm = jnp.where(base + col_ids < NUM_GENES, x_blk, 0.0)
            row_ids = jax.lax.broadcasted_iota(jnp.int32, (KBLK, 1), 0)
            wm = jnp.where(base + row_ids < NUM_GENES, w_blk, 0.0)
            acc_ref[...] = jnp.dot(
                xm, wm,
                preferred_element_type=jnp.float32,
                precision=jax.lax.Precision.DEFAULT,
            )

        @pl.when(s > 0)
        def _accum():
            acc_ref[...] += jnp.dot(
                x_blk, w_blk,
                preferred_element_type=jnp.float32,
                precision=jax.lax.Precision.DEFAULT,
            )

    @pl.when(s % 2 == 0)
    def _even():
        partial_dot(w1a_ref[...])

    @pl.when(s % 2 == 1)
    def _odd():
        partial_dot(w1b_ref[...])

    @pl.when(s == NK - 1)
    def _finish():
        h = acc_ref[...] + b1_ref[...]
        h = jnp.where(h > 0, h, 0.25 * h)
        z = jnp.dot(h, w2_ref[...], preferred_element_type=jnp.float32)
        z_ref[...] = z + b2_ref[...]


def _a_idx(s):
    return _data_blk(s - jax.lax.rem(s, 2))


def _b_idx(s):
    odd = jnp.minimum(s - jax.lax.rem(s, 2) + 1, NK - 1 - ((NK - 1) % 2 == 0))
    return _data_blk(odd)


@functools.partial(jax.jit, static_argnames=())
def kernel(x, W1, b1, W2, b2):
    b1r = b1.reshape(1, INTER_DIM)
    b2r = b2.reshape(1, LATENT_DIM)
    return pl.pallas_call(
        _mlp_kernel,
        grid=(NK,),
        in_specs=[
            pl.BlockSpec((BATCH, KBLK), lambda s: (0, _data_blk(s))),
            pl.BlockSpec((KBLK, INTER_DIM), lambda s: (_a_idx(s), 0)),
            pl.BlockSpec((KBLK, INTER_DIM), lambda s: (_b_idx(s), 0)),
            pl.BlockSpec((1, INTER_DIM), lambda s: (0, 0)),
            pl.BlockSpec((INTER_DIM, LATENT_DIM), lambda s: (0, 0)),
            pl.BlockSpec((1, LATENT_DIM), lambda s: (0, 0)),
        ],
        out_specs=pl.BlockSpec((BATCH, LATENT_DIM), lambda s: (0, 0)),
        out_shape=jax.ShapeDtypeStruct((BATCH, LATENT_DIM), jnp.float32),
        scratch_shapes=[pltpu.VMEM((BATCH, INTER_DIM), jnp.float32)],
    )(x, W1, W1, b1r, W2, b2r)


# speedup vs baseline: 1.3669x; 1.3669x over previous
import functools

import jax
import jax.numpy as jnp
from jax.experimental import pallas as pl
from jax.experimental.pallas import tpu as pltpu

NUM_GENES = 18211
INTER_DIM = 1024
LATENT_DIM = 128
BATCH = 64

KBLK = 2048
NK = (NUM_GENES + KBLK - 1) // KBLK
QUART = INTER_DIM // 4


def _mlp_kernel(x_ref, w1a_ref, w1b_ref, w1c_ref, w1d_ref, b1_ref, w2_ref,
                b2_ref, z_ref, acc_ref):
    s = pl.program_id(0)
    x_blk = x_ref[...]

    @pl.when(s == 0)
    def _first():
        base = (NK - 1) * KBLK
        col_ids = jax.lax.broadcasted_iota(jnp.int32, (BATCH, KBLK), 1)
        xm = jnp.where(base + col_ids < NUM_GENES, x_blk, 0.0)
        row_ids = jax.lax.broadcasted_iota(jnp.int32, (KBLK, 1), 0)
        rmask = base + row_ids < NUM_GENES
        for i, w_ref in enumerate((w1a_ref, w1b_ref, w1c_ref, w1d_ref)):
            acc_ref[:, i * QUART:(i + 1) * QUART] = jnp.dot(
                xm, jnp.where(rmask, w_ref[...], 0.0),
                preferred_element_type=jnp.float32,
                precision=jax.lax.Precision.DEFAULT,
            )

    @pl.when(s > 0)
    def _accum():
        for i, w_ref in enumerate((w1a_ref, w1b_ref, w1c_ref, w1d_ref)):
            acc_ref[:, i * QUART:(i + 1) * QUART] += jnp.dot(
                x_blk, w_ref[...],
                preferred_element_type=jnp.float32,
                precision=jax.lax.Precision.DEFAULT,
            )

    @pl.when(s == NK - 1)
    def _finish():
        h = acc_ref[...] + b1_ref[...]
        h = jnp.where(h > 0, h, 0.25 * h)
        z = jnp.dot(h, w2_ref[...], preferred_element_type=jnp.float32)
        z_ref[...] = z + b2_ref[...]


def _kidx(s):
    return jax.lax.rem(s + NK - 1, NK)


@functools.partial(jax.jit, static_argnames=())
def kernel(x, W1, b1, W2, b2):
    b1r = b1.reshape(1, INTER_DIM)
    b2r = b2.reshape(1, LATENT_DIM)
    return pl.pallas_call(
        _mlp_kernel,
        grid=(NK,),
        in_specs=[
            pl.BlockSpec((BATCH, KBLK), lambda s: (0, _kidx(s))),
            pl.BlockSpec((KBLK, QUART), lambda s: (_kidx(s), 0)),
            pl.BlockSpec((KBLK, QUART), lambda s: (_kidx(s), 1)),
            pl.BlockSpec((KBLK, QUART), lambda s: (_kidx(s), 2)),
            pl.BlockSpec((KBLK, QUART), lambda s: (_kidx(s), 3)),
            pl.BlockSpec((1, INTER_DIM), lambda s: (0, 0)),
            pl.BlockSpec((INTER_DIM, LATENT_DIM), lambda s: (0, 0)),
            pl.BlockSpec((1, LATENT_DIM), lambda s: (0, 0)),
        ],
        out_specs=pl.BlockSpec((BATCH, LATENT_DIM), lambda s: (0, 0)),
        out_shape=jax.ShapeDtypeStruct((BATCH, LATENT_DIM), jnp.float32),
        scratch_shapes=[pltpu.VMEM((BATCH, INTER_DIM), jnp.float32)],
    )(x, W1, W1, W1, W1, b1r, W2, b2r)
